# baseline (speedup 1.0000x reference)
import functools
import math

import jax
import jax.numpy as jnp
import numpy as np
from jax import lax
from jax.experimental import pallas as pl
from jax.experimental.pallas import tpu as pltpu
from jax.experimental.pallas import tpu_sc as plsc

NA, NT, NP, HID, NF, HEADS, HD, NL = 200, 50, 400, 128, 64, 8, 16, 2
NN = NA * NT
NPL = NP * NT
RROWS = 10240
DUMMY = 10100
WEXW = 144
TILE_E = 512
TILE_N = 1000
TILE_RAW = 400

_SNP = (np.arange(HID)[:, None] // HD == np.arange(HEADS)[None, :]).astype(np.float32)


def _ln(x, g, b):
    m = x.mean(-1, keepdims=True)
    v = ((x - m) ** 2).mean(-1, keepdims=True)
    return (x - m) * jax.lax.rsqrt(v + 1e-5) * g + b


def _wrap(a):
    two_pi = 2.0 * np.pi
    t = a + np.pi
    return t - two_pi * jnp.floor(t / two_pi) - np.pi


def _fourier_body(mode, d_in, srow, drow, freqs, w1c, w1s, w1x, b1, g1, b1n,
                  w2, b2, go, bo, wo, co, cat, post, out):
    s = srow[...]
    d = drow[...]
    go_, bo_, co_ = go[...], bo[...], co[...]
    if mode == "t" or mode == "e3":
        relx = s[:, 0:1] - s[:, 16:17]
        rely = s[:, 1:2] - s[:, 17:18]
        nrm = jnp.sqrt(relx * relx + rely * rely)
        hx = s[:, 19:20]
        hy = s[:, 20:21]
        a2 = jnp.arctan2(hx * rely - hy * relx, hx * relx + hy * rely)
        rh = _wrap(s[:, 2:3] - s[:, 18:19])
        if mode == "t":
            sd = s[:, 5:6] - s[:, 21:22]
            feats = [nrm, a2, rh, sd]
        else:
            feats = [nrm, a2, rh]
    else:
        feats = [s[:, j:j + 1] for j in range(d_in)]
    acc = jnp.zeros(out.shape, jnp.float32)
    for j in range(d_in):
        x = feats[j]
        f = x * freqs[j, :][None, :] * (2.0 * np.pi)
        h = (jnp.dot(jnp.cos(f), w1c[j], preferred_element_type=jnp.float32)
             + jnp.dot(jnp.sin(f), w1s[j], preferred_element_type=jnp.float32)
             + x * w1x[j, :][None, :] + b1[j, :][None, :])
        h = _ln(h, g1[j, :][None, :], b1n[j, :][None, :])
        h = jnp.maximum(h, 0.0)
        acc = acc + jnp.dot(h, w2[j], preferred_element_type=jnp.float32) + b2[j, :][None, :]
    if cat is not None:
        acc = acc + cat[...]
    h = _ln(acc, go_[None, :], bo_[None, :])
    h = jnp.maximum(h, 0.0)
    res = jnp.dot(h, wo[...], preferred_element_type=jnp.float32) + co_[None, :]
    if post is not None:
        res = res + post[...]
    out[...] = res


def _fourier_call(mode, srow, drow, fp, cat=None, post=None, tile=None):
    tile = (TILE_RAW if mode == 'raw' else TILE_E) if tile is None else tile
    d_in = fp['freqs'].shape[0]
    n = srow.shape[0]
    grid = (n // tile,)
    w1c = fp['W1'][:, :NF]
    w1s = fp['W1'][:, NF:2 * NF]
    w1x = fp['W1'][:, 2 * NF]
    full = lambda a: pl.BlockSpec(a.shape, lambda i: (0,) * a.ndim)
    row = lambda w: pl.BlockSpec((tile, w), lambda i: (i, 0))
    args = [srow, drow, fp['freqs'], w1c, w1s, w1x, fp['b1'], fp['g1'],
            fp['b1n'], fp['W2'], fp['b2'], fp['go'], fp['bo'], fp['Wo'], fp['co']]
    specs = [row(srow.shape[1]), row(drow.shape[1])] + [full(a) for a in args[2:]]
    if cat is not None:
        args.append(cat)
        specs.append(row(HID))
    if post is not None:
        args.append(post)
        specs.append(row(HID))
    body = functools.partial(_fourier_body, mode, d_in)
    has_cat, has_post = cat is not None, post is not None
    def body2(*refs):
        rest = list(refs[:-1])
        out_ref = refs[-1]
        post_r = rest.pop() if has_post else None
        cat_r = rest.pop() if has_cat else None
        body(*rest, cat_r, post_r, out_ref)
    return pl.pallas_call(
        body2,
        grid=grid,
        in_specs=specs,
        out_specs=pl.BlockSpec((tile, HID), lambda i: (i, 0)),
        out_shape=jax.ShapeDtypeStruct((n, HID), jnp.float32),
    )(*args)


def _dst_proj_body(x, gd, bd, wq, xd_out, q_out):
    xd = _ln(x[...], gd[...][None, :], bd[...][None, :])
    xd_out[...] = xd
    q_out[...] = jnp.dot(xd, wq[...], preferred_element_type=jnp.float32)


def _dst_proj(x, p, tile=None):
    tile = TILE_N if tile is None else tile
    n = x.shape[0]
    full = lambda a: pl.BlockSpec(a.shape, lambda i: (0,) * a.ndim)
    row = pl.BlockSpec((tile, HID), lambda i: (i, 0))
    return pl.pallas_call(
        _dst_proj_body,
        grid=(n // tile,),
        in_specs=[row, full(p['ln1g_d']), full(p['ln1b_d']), full(p['Wq'])],
        out_specs=[row, row],
        out_shape=[jax.ShapeDtypeStruct((n, HID), jnp.float32)] * 2,
    )(x, p['ln1g_d'], p['ln1b_d'], p['Wq'])


def _src_proj_body(x, gs, bs, wk, wv, kv_out):
    xs = _ln(x[...], gs[...][None, :], bs[...][None, :])
    k = jnp.dot(xs, wk[...], preferred_element_type=jnp.float32)
    v = jnp.dot(xs, wv[...], preferred_element_type=jnp.float32)
    ki = jax.lax.bitcast_convert_type(k, jnp.int32)
    vi = jax.lax.bitcast_convert_type(v, jnp.int32)
    k16 = jax.lax.shift_right_logical(ki + 0x8000, 16)
    v16 = (vi + 0x8000) & jnp.int32(-65536)
    kv_out[...] = v16 | k16


def _src_proj(x, p, tile=None):
    tile = TILE_N if tile is None else tile
    n = x.shape[0]
    full = lambda a: pl.BlockSpec(a.shape, lambda i: (0,) * a.ndim)
    return pl.pallas_call(
        _src_proj_body,
        grid=(n // tile,),
        in_specs=[pl.BlockSpec((tile, HID), lambda i: (i, 0)),
                  full(p['ln1g_s']), full(p['ln1b_s']), full(p['Wk']), full(p['Wv'])],
        out_specs=pl.BlockSpec((tile, HID), lambda i: (i, 0)),
        out_shape=jax.ShapeDtypeStruct((n, HID), jnp.int32),
    )(x, p['ln1g_s'], p['ln1b_s'], p['Wk'], p['Wv'])


def _edge_body(r, qd, kv, lg, lb, wkr, wvr, smat, stmat, out, out2):
    rr = _ln(r[...], lg[...][None, :], lb[...][None, :])
    kr = jnp.dot(rr, wkr[...], preferred_element_type=jnp.float32)
    vr = jnp.dot(rr, wvr[...], preferred_element_type=jnp.float32)
    w32 = kv[...]
    kf = jax.lax.bitcast_convert_type(jax.lax.shift_left(w32, 16), jnp.float32)
    vf = jax.lax.bitcast_convert_type(w32 & jnp.int32(-65536), jnp.float32)
    ke = kf + kr
    ve = vf + vr
    q = qd[...]
    sim = jnp.dot(q * ke, smat[...], preferred_element_type=jnp.float32) * (1.0 / math.sqrt(HD))
    ex = jnp.exp(sim)
    w = ve * jnp.dot(ex, stmat[...], preferred_element_type=jnp.float32)
    out[...] = w
    out2[...] = jnp.concatenate([ex, jnp.zeros((ex.shape[0], HID - HEADS), jnp.float32)], axis=-1)


def _edge_call(r, qd, kv, p, smat, stmat, tile=None):
    tile = TILE_E if tile is None else tile
    n = r.shape[0]
    full = lambda a: pl.BlockSpec(a.shape, lambda i: (0,) * a.ndim)
    row = lambda w: pl.BlockSpec((tile, w), lambda i: (i, 0))
    return pl.pallas_call(
        _edge_body,
        grid=(n // tile,),
        in_specs=[row(HID), row(HID), row(HID), full(p['lnrg']), full(p['lnrb']),
                  full(p['Wkr']), full(p['Wvr']), full(smat), full(stmat)],
        out_specs=[row(HID), row(HID)],
        out_shape=[jax.ShapeDtypeStruct((n, HID), jnp.float32)] * 2,
    )(r, qd, kv, p['lnrg'], p['lnrb'], p['Wkr'], p['Wvr'], smat, stmat)


def _node_body(n0, n1, d0, d1, xd, xold, wg1, wg2, bg, ws, bs, wo2, bo2,
               g2, b2, wf1, bf1, wf2, bf2, stmat, out):
    num = n0[...] + n1[...]
    den = d0[:, :HEADS] + d1[:, :HEADS]
    msg = num / (jnp.dot(den, stmat[...], preferred_element_type=jnp.float32) + 1e-16)
    xdv = xd[...]
    g = jax.nn.sigmoid(jnp.dot(msg, wg1[...], preferred_element_type=jnp.float32)
                       + jnp.dot(xdv, wg2[...], preferred_element_type=jnp.float32) + bg[...][None, :])
    agg = msg + g * (jnp.dot(xdv, ws[...], preferred_element_type=jnp.float32) + bs[...][None, :] - msg)
    x = xold[...] + jnp.dot(agg, wo2[...], preferred_element_type=jnp.float32) + bo2[...][None, :]
    h = _ln(x, g2[...][None, :], b2[...][None, :])
    h = jnp.maximum(jnp.dot(h, wf1[...], preferred_element_type=jnp.float32) + bf1[...][None, :], 0.0)
    out[...] = x + jnp.dot(h, wf2[...], preferred_element_type=jnp.float32) + bf2[...][None, :]


def _node_call(num, den, xd, xold, p, stmat, tile=None):
    tile = TILE_N if tile is None else tile
    n = xd.shape[0]
    full = lambda a: pl.BlockSpec(a.shape, lambda i: (0,) * a.ndim)
    row = lambda w: pl.BlockSpec((tile, w), lambda i: (i, 0))
    wg1 = p['Wg'][:HID]
    wg2 = p['Wg'][HID:]
    return pl.pallas_call(
        _node_body,
        grid=(n // tile,),
        in_specs=[row(HID), row(HID), row(HID), row(HID), row(HID), row(HID)]
                 + [full(a) for a in (wg1, wg2, p['bg'], p['Ws'], p['bs'],
                                      p['Wo2'], p['bo2'], p['ln2g'], p['ln2b'],
                                      p['Wf1'], p['bf1'], p['Wf2'], p['bf2'], stmat)],
        out_specs=row(HID),
        out_shape=jax.ShapeDtypeStruct((n, HID), jnp.float32),
    )(num[:n], num[RROWS:RROWS + n], den[:n], den[RROWS:RROWS + n], xd, xold, wg1, wg2, p['bg'], p['Ws'], p['bs'],
      p['Wo2'], p['bo2'], p['ln2g'], p['ln2b'], p['Wf1'], p['bf1'],
      p['Wf2'], p['bf2'], stmat)


NSC = 2
NTEC = 16
NW = NSC * NTEC
SC_CHUNK = 256
SC_CHUNK_S = 256


def _sc_gather2(tab1, idx1, tab2, idx2):
    n = idx1.shape[0]
    d1, d2 = tab1.shape[1], tab2.shape[1]
    dt1, dt2 = tab1.dtype, tab2.dtype
    b_per_w = n // NW
    nch = b_per_w // SC_CHUNK
    mesh = plsc.VectorSubcoreMesh(core_axis_name="c", subcore_axis_name="s")

    @functools.partial(
        pl.kernel, mesh=mesh,
        out_type=[jax.ShapeDtypeStruct((n, d1), dt1),
                  jax.ShapeDtypeStruct((n, d2), dt2)],
        scratch_types=[
            pltpu.VMEM((SC_CHUNK,), jnp.int32),
            pltpu.VMEM((SC_CHUNK,), jnp.int32),
            pltpu.VMEM((SC_CHUNK, d1), dt1),
            pltpu.VMEM((SC_CHUNK, d2), dt2),
            pltpu.SemaphoreType.DMA,
            pltpu.SemaphoreType.DMA,
        ],
    )
    def gk(t1, i1, t2, i2, o1, o2, iv1, iv2, r1, r2, s1, s2):
        wid = lax.axis_index("s") * NSC + lax.axis_index("c")
        base = wid * b_per_w

        def body(j, carry):
            off = base + j * SC_CHUNK
            pltpu.sync_copy(i1.at[pl.ds(off, SC_CHUNK)], iv1)
            pltpu.sync_copy(i2.at[pl.ds(off, SC_CHUNK)], iv2)
            c1 = pltpu.async_copy(t1.at[iv1], r1, s1)
            c2 = pltpu.async_copy(t2.at[iv2], r2, s2)
            c1.wait()
            c2.wait()
            pltpu.sync_copy(r1, o1.at[pl.ds(off, SC_CHUNK)])
            pltpu.sync_copy(r2, o2.at[pl.ds(off, SC_CHUNK)])
            return carry

        lax.fori_loop(0, nch, body, 0)

    return gk(tab1, idx1, tab2, idx2)


def _sc_gather_feat(tab1, idx1, tab2, idx2):
    n = idx1.shape[0]
    b_per_w = n // NW
    nch = b_per_w // SC_CHUNK
    mesh = plsc.VectorSubcoreMesh(core_axis_name="c", subcore_axis_name="s")

    @functools.partial(
        pl.kernel, mesh=mesh,
        out_type=jax.ShapeDtypeStruct((n, 32), jnp.float32),
        scratch_types=[
            pltpu.VMEM((SC_CHUNK,), jnp.int32),
            pltpu.VMEM((SC_CHUNK,), jnp.int32),
            pltpu.VMEM((SC_CHUNK, 128), jnp.float32),
            pltpu.VMEM((SC_CHUNK, 128), jnp.float32),
            pltpu.VMEM((SC_CHUNK, 32), jnp.float32),
            pltpu.SemaphoreType.DMA,
            pltpu.SemaphoreType.DMA,
        ],
    )
    def gk(t1, i1, t2, i2, o1, iv1, iv2, r1, r2, ob, s1, s2):
        wid = lax.axis_index("s") * NSC + lax.axis_index("c")
        base = wid * b_per_w

        def body(j, carry):
            off = base + j * SC_CHUNK
            pltpu.sync_copy(i1.at[pl.ds(off, SC_CHUNK)], iv1)
            pltpu.sync_copy(i2.at[pl.ds(off, SC_CHUNK)], iv2)
            c1 = pltpu.async_copy(t1.at[iv1], r1, s1)
            c2 = pltpu.async_copy(t2.at[iv2], r2, s2)
            c1.wait()
            c2.wait()

            def compact(i, carry2):
                ob[i, pl.ds(0, 16)] = r1[i, pl.ds(0, 16)]
                ob[i, pl.ds(16, 16)] = r2[i, pl.ds(0, 16)]
                return carry2

            lax.fori_loop(0, SC_CHUNK, compact, 0)
            pltpu.sync_copy(ob, o1.at[pl.ds(off, SC_CHUNK)])
            return carry

        lax.fori_loop(0, nch, body, 0)

    return gk(tab1, idx1, tab2, idx2)


def _sc_scatter_add(wex, dst3, zrows):
    n = wex.shape[0]
    nch = dst3.shape[1]
    rpt = RROWS // NTEC
    b_per_w = n // NW
    mesh = plsc.VectorSubcoreMesh(core_axis_name="c", subcore_axis_name="s")

    @functools.partial(
        pl.kernel, mesh=mesh,
        out_type=jax.ShapeDtypeStruct((NSC * RROWS, HID), jnp.float32),
        scratch_types=[
            pltpu.VMEM((SC_CHUNK_S,), jnp.int32),
            pltpu.VMEM((SC_CHUNK_S, HID), jnp.float32),
            pltpu.VMEM_SHARED((RROWS, HID), jnp.float32),
            pltpu.SemaphoreType.DMA,
            pltpu.SemaphoreType.DMA,
        ],
    )
    def sk(wex_h, idx_h, z_h, out_h, iv, wv, shared_ref, s1, s2):
        cid = lax.axis_index("c")
        sid = lax.axis_index("s")
        wid = sid * NSC + cid
        lo = sid * rpt
        pltpu.sync_copy(z_h, shared_ref.at[pl.ds(lo, rpt)])
        plsc.subcore_barrier()
        base = wid * b_per_w

        def body(j, carry):
            off = base + j * SC_CHUNK_S
            pltpu.sync_copy(idx_h.at[wid, j], iv)
            pltpu.sync_copy(wex_h.at[pl.ds(off, SC_CHUNK_S)], wv)
            pltpu.sync_copy(wv, shared_ref.at[iv], add=True)
            return carry

        lax.fori_loop(0, nch, body, 0)
        plsc.subcore_barrier()
        pltpu.sync_copy(shared_ref.at[pl.ds(lo, rpt)],
                        out_h.at[pl.ds(cid * RROWS + lo, rpt)])

    return sk(wex, dst3, zrows)


def _pad_edges(idx, n):
    pad = (-idx.shape[0]) % n
    return jnp.pad(idx, (0, pad)), idx.shape[0] + pad


def _attn_call(x_src, x_dst, r, src_p, dstg_p, dst3, zrows, p, smat, stmat):
    xd, q = _dst_proj(x_dst, p)
    kv = _src_proj(x_src, p)
    qd, kvg = _sc_gather2(q, dstg_p, kv, src_p)
    w, exr = _edge_call(r, qd, kvg, p, smat, stmat)
    num = _sc_scatter_add(w, dst3, zrows)
    den = _sc_scatter_add(exr, dst3, zrows)
    return _node_call(num, den, xd, x_dst, p, stmat)


def kernel(position, heading, velocity, pl_position, pl_orientation, x_pl,
           params, agent_type, edge_index_t, pl2a_src, pl2a_dst, a2a_src, a2a_dst):
    smat = jnp.asarray(_SNP)
    stmat = jnp.asarray(_SNP.T)

    def tm2am(s):
        return (s % NA) * NT + s // NA
    src_t = edge_index_t[0].astype(jnp.int32)
    dst_t = edge_index_t[1].astype(jnp.int32)
    p2a_s = ((pl2a_src % NP) * NT + pl2a_src // NP).astype(jnp.int32)
    p2a_d = tm2am(pl2a_dst).astype(jnp.int32)
    a2a_s = tm2am(a2a_src).astype(jnp.int32)
    a2a_d = tm2am(a2a_dst).astype(jnp.int32)

    EPAD = 16384
    edges = {}
    for name, (s, d) in dict(t=(src_t, dst_t), p=(p2a_s, p2a_d), a=(a2a_s, a2a_d)).items():
        sp, e = _pad_edges(s, EPAD)
        dg, _ = _pad_edges(d, EPAD)
        npad = e - s.shape[0]
        ds_ = jnp.concatenate([d, jnp.full((npad,), DUMMY, jnp.int32)])
        dst3 = ds_.reshape(NW, e // NW // SC_CHUNK_S, SC_CHUNK_S)
        edges[name] = (sp, dg, dst3, e)

    hv = jnp.stack([jnp.cos(heading), jnp.sin(heading)], axis=-1)
    ids = jnp.arange(NN, dtype=jnp.float32)
    tt = jnp.concatenate([position.reshape(-1, 2), heading.reshape(-1, 1),
                          hv.reshape(-1, 2), ids[:, None],
                          jnp.zeros((NN, 122), jnp.float32)], axis=-1)
    plt = jnp.concatenate([jnp.repeat(pl_position, NT, axis=0),
                           jnp.repeat(pl_orientation, NT)[:, None],
                           jnp.zeros((NPL, 125), jnp.float32)], axis=-1)

    st_, dt_, _, _ = edges['t']
    sd_t = _sc_gather_feat(tt, st_, tt, dt_)
    r_t = _fourier_call("t", sd_t, sd_t, params['r_t_emb'])
    sp_, dp_, _, _ = edges['p']
    sd_p = _sc_gather_feat(plt, sp_, tt, dp_)
    r_p = _fourier_call("e3", sd_p, sd_p, params['r_pl2a_emb'])
    sa_, da_, _, _ = edges['a']
    sd_a = _sc_gather_feat(tt, sa_, tt, da_)
    r_a = _fourier_call("e3", sd_a, sd_a, params['r_a2a_emb'])

    motion = jnp.concatenate([jnp.zeros((NA, 1, 2), jnp.float32),
                              position[:, 1:] - position[:, :-1]], axis=1)

    def ang2d(ctr, nbr):
        return jnp.arctan2(ctr[..., 0] * nbr[..., 1] - ctr[..., 1] * nbr[..., 0],
                           (ctr * nbr).sum(-1))
    x_feat = jnp.stack([
        jnp.linalg.norm(motion, axis=-1),
        ang2d(hv, motion),
        jnp.linalg.norm(velocity, axis=-1),
        ang2d(hv, velocity)], axis=-1).reshape(-1, 4)
    x_feat = jnp.pad(x_feat, ((0, 0), (0, 12)))
    pos = jnp.arange(NT, dtype=jnp.float32)[:, None]
    i2 = jnp.arange(HID // 2, dtype=jnp.float32)[None, :]
    ang = pos / jnp.power(10000.0, 2.0 * i2 / HID)
    pe = jnp.zeros((NT, HID)).at[:, 0::2].set(jnp.sin(ang)).at[:, 1::2].set(jnp.cos(ang))
    cat_emb = jnp.repeat(params['type_emb'][agent_type.astype(jnp.int32)], NT, axis=0)
    pe_full = jnp.tile(pe, (NA, 1))
    x = _fourier_call("raw", x_feat, x_feat, params['x_a_emb'], cat=cat_emb, post=pe_full)

    x_pl_f = x_pl.reshape(NPL, HID)
    zrows = jnp.zeros((RROWS // NTEC, HID), jnp.float32)

    for i in range(NL):
        st_, dg_, ds_, _ = edges['t']
        x = _attn_call(x, x, r_t, st_, dg_, ds_, zrows, params['t_layers'][i], smat, stmat)
        sp_, dg_, ds_, _ = edges['p']
        x = _attn_call(x_pl_f, x, r_p, sp_, dg_, ds_, zrows, params['pl2a_layers'][i], smat, stmat)
        sa_, dg_, ds_, _ = edges['a']
        x = _attn_call(x, x, r_a, sa_, dg_, ds_, zrows, params['a2a_layers'][i], smat, stmat)
    return x.reshape(NA, NT, HID)


# speedup vs baseline: 1.2309x; 1.2309x over previous
import functools
import math

import jax
import jax.numpy as jnp
import numpy as np
from jax import lax
from jax.experimental import pallas as pl
from jax.experimental.pallas import tpu as pltpu
from jax.experimental.pallas import tpu_sc as plsc

NA, NT, NP, HID, NF, HEADS, HD, NL = 200, 50, 400, 128, 64, 8, 16, 2
NN = NA * NT
NPL = NP * NT
RROWS = 10240
DUMMY = 10100
WEXW = 144
TILE_E = 512
TILE_N = 1000
TILE_RAW = 400

_SNP = (np.arange(HID)[:, None] // HD == np.arange(HEADS)[None, :]).astype(np.float32)


def _ln(x, g, b):
    m = x.mean(-1, keepdims=True)
    v = ((x - m) ** 2).mean(-1, keepdims=True)
    return (x - m) * jax.lax.rsqrt(v + 1e-5) * g + b


def _wrap(a):
    two_pi = 2.0 * np.pi
    t = a + np.pi
    return t - two_pi * jnp.floor(t / two_pi) - np.pi


def _fourier_body(mode, d_in, srow, drow, freqs, w1c, w1s, w1x, b1, g1, b1n,
                  w2, b2, go, bo, wo, co, cat, post, out):
    s = srow[...]
    d = drow[...]
    go_, bo_, co_ = go[...], bo[...], co[...]
    if mode == "t" or mode == "e3":
        relx = s[:, 0:1] - s[:, 16:17]
        rely = s[:, 1:2] - s[:, 17:18]
        nrm = jnp.sqrt(relx * relx + rely * rely)
        hx = s[:, 19:20]
        hy = s[:, 20:21]
        a2 = jnp.arctan2(hx * rely - hy * relx, hx * relx + hy * rely)
        rh = _wrap(s[:, 2:3] - s[:, 18:19])
        if mode == "t":
            sd = s[:, 5:6] - s[:, 21:22]
            feats = [nrm, a2, rh, sd]
        else:
            feats = [nrm, a2, rh]
    else:
        feats = [s[:, j:j + 1] for j in range(d_in)]
    acc = jnp.zeros(out.shape, jnp.float32)
    for j in range(d_in):
        x = feats[j]
        f = x * freqs[j, :][None, :] * (2.0 * np.pi)
        h = (jnp.dot(jnp.cos(f).astype(jnp.bfloat16), w1c[j].astype(jnp.bfloat16),
                     preferred_element_type=jnp.float32)
             + jnp.dot(jnp.sin(f).astype(jnp.bfloat16), w1s[j].astype(jnp.bfloat16),
                       preferred_element_type=jnp.float32)
             + x * w1x[j, :][None, :] + b1[j, :][None, :])
        h = _ln(h, g1[j, :][None, :], b1n[j, :][None, :])
        h = jnp.maximum(h, 0.0)
        acc = acc + jnp.dot(h.astype(jnp.bfloat16), w2[j].astype(jnp.bfloat16),
                            preferred_element_type=jnp.float32) + b2[j, :][None, :]
    if cat is not None:
        acc = acc + cat[...]
    h = _ln(acc, go_[None, :], bo_[None, :])
    h = jnp.maximum(h, 0.0)
    res = jnp.dot(h.astype(jnp.bfloat16), wo[...].astype(jnp.bfloat16),
                  preferred_element_type=jnp.float32) + co_[None, :]
    if post is not None:
        res = res + post[...]
    out[...] = res


def _fourier_call(mode, srow, drow, fp, cat=None, post=None, tile=None):
    tile = (TILE_RAW if mode == 'raw' else TILE_E) if tile is None else tile
    d_in = fp['freqs'].shape[0]
    n = srow.shape[0]
    grid = (n // tile,)
    w1c = fp['W1'][:, :NF]
    w1s = fp['W1'][:, NF:2 * NF]
    w1x = fp['W1'][:, 2 * NF]
    full = lambda a: pl.BlockSpec(a.shape, lambda i: (0,) * a.ndim)
    row = lambda w: pl.BlockSpec((tile, w), lambda i: (i, 0))
    args = [srow, drow, fp['freqs'], w1c, w1s, w1x, fp['b1'], fp['g1'],
            fp['b1n'], fp['W2'], fp['b2'], fp['go'], fp['bo'], fp['Wo'], fp['co']]
    specs = [row(srow.shape[1]), row(drow.shape[1])] + [full(a) for a in args[2:]]
    if cat is not None:
        args.append(cat)
        specs.append(row(HID))
    if post is not None:
        args.append(post)
        specs.append(row(HID))
    body = functools.partial(_fourier_body, mode, d_in)
    has_cat, has_post = cat is not None, post is not None
    def body2(*refs):
        rest = list(refs[:-1])
        out_ref = refs[-1]
        post_r = rest.pop() if has_post else None
        cat_r = rest.pop() if has_cat else None
        body(*rest, cat_r, post_r, out_ref)
    return pl.pallas_call(
        body2,
        grid=grid,
        in_specs=specs,
        out_specs=pl.BlockSpec((tile, HID), lambda i: (i, 0)),
        out_shape=jax.ShapeDtypeStruct((n, HID), jnp.float32),
    )(*args)


def _dst_proj_body(x, gd, bd, wq, xd_out, q_out):
    xd = _ln(x[...], gd[...][None, :], bd[...][None, :])
    xd_out[...] = xd
    q_out[...] = jnp.dot(xd, wq[...], preferred_element_type=jnp.float32)


def _dst_proj(x, p, tile=None):
    tile = TILE_N if tile is None else tile
    n = x.shape[0]
    full = lambda a: pl.BlockSpec(a.shape, lambda i: (0,) * a.ndim)
    row = pl.BlockSpec((tile, HID), lambda i: (i, 0))
    return pl.pallas_call(
        _dst_proj_body,
        grid=(n // tile,),
        in_specs=[row, full(p['ln1g_d']), full(p['ln1b_d']), full(p['Wq'])],
        out_specs=[row, row],
        out_shape=[jax.ShapeDtypeStruct((n, HID), jnp.float32)] * 2,
    )(x, p['ln1g_d'], p['ln1b_d'], p['Wq'])


def _src_proj_body(x, gs, bs, wk, wv, kv_out):
    xs = _ln(x[...], gs[...][None, :], bs[...][None, :])
    k = jnp.dot(xs, wk[...], preferred_element_type=jnp.float32)
    v = jnp.dot(xs, wv[...], preferred_element_type=jnp.float32)
    ki = jax.lax.bitcast_convert_type(k, jnp.int32)
    vi = jax.lax.bitcast_convert_type(v, jnp.int32)
    k16 = jax.lax.shift_right_logical(ki + 0x8000, 16)
    v16 = (vi + 0x8000) & jnp.int32(-65536)
    kv_out[...] = v16 | k16


def _src_proj(x, p, tile=None):
    tile = TILE_N if tile is None else tile
    n = x.shape[0]
    full = lambda a: pl.BlockSpec(a.shape, lambda i: (0,) * a.ndim)
    return pl.pallas_call(
        _src_proj_body,
        grid=(n // tile,),
        in_specs=[pl.BlockSpec((tile, HID), lambda i: (i, 0)),
                  full(p['ln1g_s']), full(p['ln1b_s']), full(p['Wk']), full(p['Wv'])],
        out_specs=pl.BlockSpec((tile, HID), lambda i: (i, 0)),
        out_shape=jax.ShapeDtypeStruct((n, HID), jnp.int32),
    )(x, p['ln1g_s'], p['ln1b_s'], p['Wk'], p['Wv'])


def _edge_body(r, qd, kv, lg, lb, wkr, wvr, smat, stmat, out, out2):
    rr = _ln(r[...], lg[...][None, :], lb[...][None, :]).astype(jnp.bfloat16)
    kr = jnp.dot(rr, wkr[...].astype(jnp.bfloat16), preferred_element_type=jnp.float32)
    vr = jnp.dot(rr, wvr[...].astype(jnp.bfloat16), preferred_element_type=jnp.float32)
    w32 = kv[...]
    kf = jax.lax.bitcast_convert_type(jax.lax.shift_left(w32, 16), jnp.float32)
    vf = jax.lax.bitcast_convert_type(w32 & jnp.int32(-65536), jnp.float32)
    ke = kf + kr
    ve = vf + vr
    q = qd[...]
    sim = jnp.dot(q * ke, smat[...], preferred_element_type=jnp.float32) * (1.0 / math.sqrt(HD))
    ex = jnp.exp(sim)
    w = ve * jnp.dot(ex, stmat[...], preferred_element_type=jnp.float32)
    out[...] = w
    out2[...] = jnp.concatenate([ex, jnp.zeros((ex.shape[0], HID - HEADS), jnp.float32)], axis=-1)


def _edge_call(r, qd, kv, p, smat, stmat, tile=None):
    tile = TILE_E if tile is None else tile
    n = r.shape[0]
    full = lambda a: pl.BlockSpec(a.shape, lambda i: (0,) * a.ndim)
    row = lambda w: pl.BlockSpec((tile, w), lambda i: (i, 0))
    return pl.pallas_call(
        _edge_body,
        grid=(n // tile,),
        in_specs=[row(HID), row(HID), row(HID), full(p['lnrg']), full(p['lnrb']),
                  full(p['Wkr']), full(p['Wvr']), full(smat), full(stmat)],
        out_specs=[row(HID), row(HID)],
        out_shape=[jax.ShapeDtypeStruct((n, HID), jnp.float32)] * 2,
    )(r, qd, kv, p['lnrg'], p['lnrb'], p['Wkr'], p['Wvr'], smat, stmat)


def _node_body(n0, n1, d0, d1, xd, xold, wg1, wg2, bg, ws, bs, wo2, bo2,
               g2, b2, wf1, bf1, wf2, bf2, stmat, out):
    num = n0[...] + n1[...]
    den = d0[:, :HEADS] + d1[:, :HEADS]
    msg = num / (jnp.dot(den, stmat[...], preferred_element_type=jnp.float32) + 1e-16)
    xdv = xd[...]
    g = jax.nn.sigmoid(jnp.dot(msg, wg1[...], preferred_element_type=jnp.float32)
                       + jnp.dot(xdv, wg2[...], preferred_element_type=jnp.float32) + bg[...][None, :])
    agg = msg + g * (jnp.dot(xdv, ws[...], preferred_element_type=jnp.float32) + bs[...][None, :] - msg)
    x = xold[...] + jnp.dot(agg, wo2[...], preferred_element_type=jnp.float32) + bo2[...][None, :]
    h = _ln(x, g2[...][None, :], b2[...][None, :])
    h = jnp.maximum(jnp.dot(h, wf1[...], preferred_element_type=jnp.float32) + bf1[...][None, :], 0.0)
    out[...] = x + jnp.dot(h, wf2[...], preferred_element_type=jnp.float32) + bf2[...][None, :]


def _node_call(num, den, xd, xold, p, stmat, tile=None):
    tile = TILE_N if tile is None else tile
    n = xd.shape[0]
    full = lambda a: pl.BlockSpec(a.shape, lambda i: (0,) * a.ndim)
    row = lambda w: pl.BlockSpec((tile, w), lambda i: (i, 0))
    wg1 = p['Wg'][:HID]
    wg2 = p['Wg'][HID:]
    return pl.pallas_call(
        _node_body,
        grid=(n // tile,),
        in_specs=[row(HID), row(HID), row(HID), row(HID), row(HID), row(HID)]
                 + [full(a) for a in (wg1, wg2, p['bg'], p['Ws'], p['bs'],
                                      p['Wo2'], p['bo2'], p['ln2g'], p['ln2b'],
                                      p['Wf1'], p['bf1'], p['Wf2'], p['bf2'], stmat)],
        out_specs=row(HID),
        out_shape=jax.ShapeDtypeStruct((n, HID), jnp.float32),
    )(num[:n], num[RROWS:RROWS + n], den[:n], den[RROWS:RROWS + n], xd, xold, wg1, wg2, p['bg'], p['Ws'], p['bs'],
      p['Wo2'], p['bo2'], p['ln2g'], p['ln2b'], p['Wf1'], p['bf1'],
      p['Wf2'], p['bf2'], stmat)


NSC = 2
NTEC = 16
NW = NSC * NTEC
SC_CHUNK = 128
SC_CHUNK_S = 128


def _sc_gather2(tab1, idx1, tab2, idx2):
    n = idx1.shape[0]
    d1, d2 = tab1.shape[1], tab2.shape[1]
    dt1, dt2 = tab1.dtype, tab2.dtype
    b_per_w = n // NW
    nch = b_per_w // SC_CHUNK
    mesh = plsc.VectorSubcoreMesh(core_axis_name="c", subcore_axis_name="s")

    @functools.partial(
        pl.kernel, mesh=mesh,
        out_type=[jax.ShapeDtypeStruct((n, d1), dt1),
                  jax.ShapeDtypeStruct((n, d2), dt2)],
        scratch_types=[
            pltpu.VMEM((SC_CHUNK,), jnp.int32),
            pltpu.VMEM((SC_CHUNK,), jnp.int32),
            pltpu.VMEM((SC_CHUNK, d1), dt1),
            pltpu.VMEM((SC_CHUNK, d2), dt2),
            pltpu.SemaphoreType.DMA,
            pltpu.SemaphoreType.DMA,
        ],
    )
    def gk(t1, i1, t2, i2, o1, o2, iv1, iv2, r1, r2, s1, s2):
        wid = lax.axis_index("s") * NSC + lax.axis_index("c")
        base = wid * b_per_w

        def body(j, carry):
            off = base + j * SC_CHUNK
            pltpu.sync_copy(i1.at[pl.ds(off, SC_CHUNK)], iv1)
            pltpu.sync_copy(i2.at[pl.ds(off, SC_CHUNK)], iv2)
            c1 = pltpu.async_copy(t1.at[iv1], r1, s1)
            c2 = pltpu.async_copy(t2.at[iv2], r2, s2)
            c1.wait()
            c2.wait()
            pltpu.sync_copy(r1, o1.at[pl.ds(off, SC_CHUNK)])
            pltpu.sync_copy(r2, o2.at[pl.ds(off, SC_CHUNK)])
            return carry

        lax.fori_loop(0, nch, body, 0)

    return gk(tab1, idx1, tab2, idx2)


def _sc_gather_feat(tab1, idx1, tab2, idx2):
    n = idx1.shape[0]
    b_per_w = n // NW
    nch = b_per_w // SC_CHUNK
    mesh = plsc.VectorSubcoreMesh(core_axis_name="c", subcore_axis_name="s")

    @functools.partial(
        pl.kernel, mesh=mesh,
        out_type=jax.ShapeDtypeStruct((n, 32), jnp.float32),
        scratch_types=[
            pltpu.VMEM((SC_CHUNK,), jnp.int32),
            pltpu.VMEM((SC_CHUNK,), jnp.int32),
            pltpu.VMEM((SC_CHUNK, 128), jnp.float32),
            pltpu.VMEM((SC_CHUNK, 128), jnp.float32),
            pltpu.VMEM((SC_CHUNK, 32), jnp.float32),
            pltpu.SemaphoreType.DMA,
            pltpu.SemaphoreType.DMA,
        ],
    )
    def gk(t1, i1, t2, i2, o1, iv1, iv2, r1, r2, ob, s1, s2):
        wid = lax.axis_index("s") * NSC + lax.axis_index("c")
        base = wid * b_per_w

        def body(j, carry):
            off = base + j * SC_CHUNK
            pltpu.sync_copy(i1.at[pl.ds(off, SC_CHUNK)], iv1)
            pltpu.sync_copy(i2.at[pl.ds(off, SC_CHUNK)], iv2)
            c1 = pltpu.async_copy(t1.at[iv1], r1, s1)
            c2 = pltpu.async_copy(t2.at[iv2], r2, s2)
            c1.wait()
            c2.wait()

            def compact(i, carry2):
                ob[i, pl.ds(0, 16)] = r1[i, pl.ds(0, 16)]
                ob[i, pl.ds(16, 16)] = r2[i, pl.ds(0, 16)]
                return carry2

            lax.fori_loop(0, SC_CHUNK, compact, 0)
            pltpu.sync_copy(ob, o1.at[pl.ds(off, SC_CHUNK)])
            return carry

        lax.fori_loop(0, nch, body, 0)

    return gk(tab1, idx1, tab2, idx2)


def _sc_scatter_add(wex, dst3, zrows):
    n = wex.shape[0]
    nch = dst3.shape[1]
    rpt = RROWS // NTEC
    b_per_w = n // NW
    mesh = plsc.VectorSubcoreMesh(core_axis_name="c", subcore_axis_name="s")

    @functools.partial(
        pl.kernel, mesh=mesh,
        out_type=jax.ShapeDtypeStruct((NSC * RROWS, HID), jnp.float32),
        scratch_types=[
            pltpu.VMEM((SC_CHUNK_S,), jnp.int32),
            pltpu.VMEM((SC_CHUNK_S, HID), jnp.float32),
            pltpu.VMEM_SHARED((RROWS, HID), jnp.float32),
            pltpu.SemaphoreType.DMA,
            pltpu.SemaphoreType.DMA,
        ],
    )
    def sk(wex_h, idx_h, z_h, out_h, iv, wv, shared_ref, s1, s2):
        cid = lax.axis_index("c")
        sid = lax.axis_index("s")
        wid = sid * NSC + cid
        lo = sid * rpt
        pltpu.sync_copy(z_h, shared_ref.at[pl.ds(lo, rpt)])
        plsc.subcore_barrier()
        base = wid * b_per_w

        def body(j, carry):
            off = base + j * SC_CHUNK_S
            pltpu.sync_copy(idx_h.at[wid, j], iv)
            pltpu.sync_copy(wex_h.at[pl.ds(off, SC_CHUNK_S)], wv)
            pltpu.sync_copy(wv, shared_ref.at[iv], add=True)
            return carry

        lax.fori_loop(0, nch, body, 0)
        plsc.subcore_barrier()
        pltpu.sync_copy(shared_ref.at[pl.ds(lo, rpt)],
                        out_h.at[pl.ds(cid * RROWS + lo, rpt)])

    return sk(wex, dst3, zrows)


def _pad_edges(idx, n):
    pad = (-idx.shape[0]) % n
    return jnp.pad(idx, (0, pad)), idx.shape[0] + pad


def _attn_call(x_src, x_dst, r, src_p, dstg_p, dst3, zrows, p, smat, stmat):
    xd, q = _dst_proj(x_dst, p)
    kv = _src_proj(x_src, p)
    qd, kvg = _sc_gather2(q, dstg_p, kv, src_p)
    w, exr = _edge_call(r, qd, kvg, p, smat, stmat)
    num = _sc_scatter_add(w, dst3, zrows)
    den = _sc_scatter_add(exr, dst3, zrows)
    return _node_call(num, den, xd, x_dst, p, stmat)


def kernel(position, heading, velocity, pl_position, pl_orientation, x_pl,
           params, agent_type, edge_index_t, pl2a_src, pl2a_dst, a2a_src, a2a_dst):
    smat = jnp.asarray(_SNP)
    stmat = jnp.asarray(_SNP.T)

    def tm2am(s):
        return (s % NA) * NT + s // NA
    src_t = edge_index_t[0].astype(jnp.int32)
    dst_t = edge_index_t[1].astype(jnp.int32)
    p2a_s = ((pl2a_src % NP) * NT + pl2a_src // NP).astype(jnp.int32)
    p2a_d = tm2am(pl2a_dst).astype(jnp.int32)
    a2a_s = tm2am(a2a_src).astype(jnp.int32)
    a2a_d = tm2am(a2a_dst).astype(jnp.int32)

    EPAD = 4096
    edges = {}
    for name, (s, d) in dict(t=(src_t, dst_t), p=(p2a_s, p2a_d), a=(a2a_s, a2a_d)).items():
        sp, e = _pad_edges(s, EPAD)
        dg, _ = _pad_edges(d, EPAD)
        npad = e - s.shape[0]
        ds_ = jnp.concatenate([d, jnp.full((npad,), DUMMY, jnp.int32)])
        dst3 = ds_.reshape(NW, e // NW // SC_CHUNK_S, SC_CHUNK_S)
        edges[name] = (sp, dg, dst3, e)

    hv = jnp.stack([jnp.cos(heading), jnp.sin(heading)], axis=-1)
    ids = jnp.arange(NN, dtype=jnp.float32)
    tt = jnp.concatenate([position.reshape(-1, 2), heading.reshape(-1, 1),
                          hv.reshape(-1, 2), ids[:, None],
                          jnp.zeros((NN, 122), jnp.float32)], axis=-1)
    plt = jnp.concatenate([jnp.repeat(pl_position, NT, axis=0),
                           jnp.repeat(pl_orientation, NT)[:, None],
                           jnp.zeros((NPL, 125), jnp.float32)], axis=-1)

    st_, dt_, _, _ = edges['t']
    sd_t = _sc_gather_feat(tt, st_, tt, dt_)
    r_t = _fourier_call("t", sd_t, sd_t, params['r_t_emb'])
    sp_, dp_, _, _ = edges['p']
    sd_p = _sc_gather_feat(plt, sp_, tt, dp_)
    r_p = _fourier_call("e3", sd_p, sd_p, params['r_pl2a_emb'])
    sa_, da_, _, _ = edges['a']
    sd_a = _sc_gather_feat(tt, sa_, tt, da_)
    r_a = _fourier_call("e3", sd_a, sd_a, params['r_a2a_emb'])

    motion = jnp.concatenate([jnp.zeros((NA, 1, 2), jnp.float32),
                              position[:, 1:] - position[:, :-1]], axis=1)

    def ang2d(ctr, nbr):
        return jnp.arctan2(ctr[..., 0] * nbr[..., 1] - ctr[..., 1] * nbr[..., 0],
                           (ctr * nbr).sum(-1))
    x_feat = jnp.stack([
        jnp.linalg.norm(motion, axis=-1),
        ang2d(hv, motion),
        jnp.linalg.norm(velocity, axis=-1),
        ang2d(hv, velocity)], axis=-1).reshape(-1, 4)
    x_feat = jnp.pad(x_feat, ((0, 0), (0, 12)))
    pos = jnp.arange(NT, dtype=jnp.float32)[:, None]
    i2 = jnp.arange(HID // 2, dtype=jnp.float32)[None, :]
    ang = pos / jnp.power(10000.0, 2.0 * i2 / HID)
    pe = jnp.zeros((NT, HID)).at[:, 0::2].set(jnp.sin(ang)).at[:, 1::2].set(jnp.cos(ang))
    cat_emb = jnp.repeat(params['type_emb'][agent_type.astype(jnp.int32)], NT, axis=0)
    pe_full = jnp.tile(pe, (NA, 1))
    x = _fourier_call("raw", x_feat, x_feat, params['x_a_emb'], cat=cat_emb, post=pe_full)

    x_pl_f = x_pl.reshape(NPL, HID)
    zrows = jnp.zeros((RROWS // NTEC, HID), jnp.float32)

    for i in range(NL):
        st_, dg_, ds_, _ = edges['t']
        x = _attn_call(x, x, r_t, st_, dg_, ds_, zrows, params['t_layers'][i], smat, stmat)
        sp_, dg_, ds_, _ = edges['p']
        x = _attn_call(x_pl_f, x, r_p, sp_, dg_, ds_, zrows, params['pl2a_layers'][i], smat, stmat)
        sa_, dg_, ds_, _ = edges['a']
        x = _attn_call(x, x, r_a, sa_, dg_, ds_, zrows, params['a2a_layers'][i], smat, stmat)
    return x.reshape(NA, NT, HID)
